# Initial kernel scaffold; baseline (speedup 1.0000x reference)
import functools

import jax
import jax.numpy as jnp
from jax import lax
from jax.experimental import pallas as pl
from jax.experimental.pallas import tpu as pltpu
from jax.experimental.pallas import tpu_sc as plsc

_N = 10000
_E = 160000
_D = 256
_HALF = 128
_NC = 2
_NS = 16
_CHUNK = 128
_EPAD = 2048 * 80
_CPT = _EPAD // (_NS * _CHUNK)
_NRC = _EPAD // _CHUNK
_AGG_ROWS = 10240
_DUMMY = _N
_DEGW = 16
_BN = 1000


def _make_mesh():
    return plsc.VectorSubcoreMesh(
        core_axis_name="c", subcore_axis_name="s",
        num_cores=_NC, num_subcores=_NS)


@functools.partial(
    pl.kernel,
    out_type=jax.ShapeDtypeStruct((_NC * _AGG_ROWS, _DEGW), jnp.float32),
    mesh=_make_mesh(),
    scratch_types=[
        pltpu.VMEM((_CPT, _CHUNK), jnp.int32),
        pltpu.VMEM((_CHUNK, _DEGW), jnp.float32),
        pltpu.VMEM((40, _DEGW), jnp.float32),
        pltpu.VMEM_SHARED((_AGG_ROWS, _DEGW), jnp.float32),
    ],
)
def _deg_kernel(eidx_hbm, deg_hbm, didx, ones, zb, deg):
    c = lax.axis_index("c")
    t = lax.axis_index("s")
    pltpu.sync_copy(eidx_hbm.at[pl.ds(c * _NRC + t * _CPT, _CPT)], didx)
    one = jnp.ones((16,), jnp.float32)
    zero = jnp.zeros((16,), jnp.float32)
    for i in range(_CHUNK):
        ones[i, :] = one
    for i in range(40):
        zb[i, :] = zero
    rpt = _AGG_ROWS // _NS

    def zero_body(k, carry):
        pltpu.sync_copy(zb, deg.at[pl.ds(t * rpt + k * 40, 40)])
        return carry

    lax.fori_loop(0, rpt // 40, zero_body, 0)
    plsc.subcore_barrier()

    def body(g, carry):
        pltpu.sync_copy(ones, deg.at[didx.at[g]], add=True)
        return carry

    lax.fori_loop(0, _CPT, body, 0)
    plsc.subcore_barrier()
    pltpu.sync_copy(deg.at[pl.ds(t * rpt, rpt)],
                    deg_hbm.at[pl.ds(c * _AGG_ROWS + t * rpt, rpt)])


@functools.partial(
    pl.kernel,
    out_type=jax.ShapeDtypeStruct((_NC * _N, _HALF), jnp.float32),
    mesh=_make_mesh(),
    scratch_types=[
        pltpu.VMEM((_CPT, _CHUNK), jnp.int32),
        pltpu.VMEM((_CPT, _CHUNK), jnp.int32),
        pltpu.VMEM((_CHUNK, _HALF), jnp.float32),
        pltpu.VMEM((_CHUNK, _HALF), jnp.float32),
        pltpu.VMEM((16, _HALF), jnp.float32),
        pltpu.VMEM_SHARED((_AGG_ROWS, _HALF), jnp.float32),
        pltpu.SemaphoreType.DMA,
        pltpu.SemaphoreType.DMA,
    ],
)
def _agg_kernel(x_hbm, gidx_hbm, sidx_hbm, out_hbm,
                gidx, didx, buf0, buf1, zbuf, agg, sem0, sem1):
    c = lax.axis_index("c")
    t = lax.axis_index("s")
    pltpu.sync_copy(gidx_hbm.at[pl.ds(c * _NRC + t * _CPT, _CPT)], gidx)
    pltpu.sync_copy(sidx_hbm.at[pl.ds(t * _CPT, _CPT)], didx)

    zero = jnp.zeros((16,), jnp.float32)
    for i in range(16):
        for j in range(_HALF // 16):
            zbuf[i, pl.ds(j * 16, 16)] = zero
    rpt = _AGG_ROWS // _NS

    def zero_body(k, carry):
        pltpu.sync_copy(zbuf, agg.at[pl.ds(t * rpt + k * 16, 16)])
        return carry

    lax.fori_loop(0, rpt // 16, zero_body, 0)
    plsc.subcore_barrier()

    def gather(g, buf, sem):
        return pltpu.make_async_copy(x_hbm.at[gidx.at[g]], buf, sem)

    gather(0, buf0, sem0).start()

    def body(i, carry):
        g0 = 2 * i
        g1 = 2 * i + 1
        gather(g0, buf0, sem0).wait()
        gather(g1, buf1, sem1).start()
        pltpu.sync_copy(buf0, agg.at[didx.at[g0]], add=True)
        gather(g1, buf1, sem1).wait()

        @pl.when(i < _CPT // 2 - 1)
        def _():
            gather(g1 + 1, buf0, sem0).start()

        pltpu.sync_copy(buf1, agg.at[didx.at[g1]], add=True)
        return carry

    lax.fori_loop(0, _CPT // 2, body, 0)
    plsc.subcore_barrier()
    orows = _N // _NS
    pltpu.sync_copy(agg.at[pl.ds(t * orows, orows)],
                    out_hbm.at[pl.ds(c * _N + t * orows, orows)])


def _mm1(h, cnt_out, W1):
    def body(h_ref, c_ref, w_ref, o_ref):
        s = lax.rsqrt(jnp.maximum(c_ref[...], 1.0))
        r = jnp.dot(h_ref[...] * s, w_ref[...],
                    preferred_element_type=jnp.float32)
        o_ref[0] = r[:, :_HALF]
        o_ref[1] = r[:, _HALF:]

    return pl.pallas_call(
        body,
        grid=(_N // _BN,),
        in_specs=[
            pl.BlockSpec((_BN, _D), lambda i: (i, 0)),
            pl.BlockSpec((_BN, 1), lambda i: (i, 0)),
            pl.BlockSpec((_D, _D), lambda i: (0, 0)),
        ],
        out_specs=pl.BlockSpec((_NC, _BN, _HALF), lambda i: (0, i, 0)),
        out_shape=jax.ShapeDtypeStruct((_NC, _N, _HALF), jnp.float32),
    )(h, cnt_out, W1)


def _mm2(agg1, cnt_in, cnt_out, b1, W2):
    nb = _N // _BN

    def body(lo_ref, hi_ref, ci_ref, co_ref, b_ref, w_ref, o_ref):
        s_in = lax.rsqrt(jnp.maximum(ci_ref[...], 1.0))
        s_out = lax.rsqrt(jnp.maximum(co_ref[...], 1.0))
        a = jnp.concatenate([lo_ref[...], hi_ref[...]], axis=1)
        h1 = jnp.maximum(a * s_in + b_ref[...], 0.0)
        r = jnp.dot(h1 * s_out, w_ref[...], preferred_element_type=jnp.float32)
        o_ref[0] = r[:, :_HALF]
        o_ref[1] = r[:, _HALF:]

    return pl.pallas_call(
        body,
        grid=(nb,),
        in_specs=[
            pl.BlockSpec((_BN, _HALF), lambda i: (i, 0)),
            pl.BlockSpec((_BN, _HALF), lambda i: (nb + i, 0)),
            pl.BlockSpec((_BN, 1), lambda i: (i, 0)),
            pl.BlockSpec((_BN, 1), lambda i: (i, 0)),
            pl.BlockSpec((1, _D), lambda i: (0, 0)),
            pl.BlockSpec((_D, _D), lambda i: (0, 0)),
        ],
        out_specs=pl.BlockSpec((_NC, _BN, _HALF), lambda i: (0, i, 0)),
        out_shape=jax.ShapeDtypeStruct((_NC, _N, _HALF), jnp.float32),
    )(agg1, agg1, cnt_in, cnt_out, b1, W2)


def _final(agg2, cnt_in, b2):
    nb = _N // _BN

    def body(lo_ref, hi_ref, ci_ref, b_ref, o_ref):
        s_in = lax.rsqrt(jnp.maximum(ci_ref[...], 1.0))
        a = jnp.concatenate([lo_ref[...], hi_ref[...]], axis=1)
        o_ref[...] = a * s_in + b_ref[...]

    return pl.pallas_call(
        body,
        grid=(nb,),
        in_specs=[
            pl.BlockSpec((_BN, _HALF), lambda i: (i, 0)),
            pl.BlockSpec((_BN, _HALF), lambda i: (nb + i, 0)),
            pl.BlockSpec((_BN, 1), lambda i: (i, 0)),
            pl.BlockSpec((1, _D), lambda i: (0, 0)),
        ],
        out_specs=pl.BlockSpec((_BN, _D), lambda i: (i, 0)),
        out_shape=jax.ShapeDtypeStruct((_N, _D), jnp.float32),
    )(agg2, agg2, cnt_in, b2)


def kernel(h, edge_index, W1, b1, W2, b2):
    src = edge_index[0]
    dst = edge_index[1]
    pad = _EPAD - _E
    src_p = jnp.concatenate([src, jnp.zeros((pad,), jnp.int32)])
    dst_p = jnp.concatenate([dst, jnp.full((pad,), _DUMMY, jnp.int32)])
    gather_idx = jnp.stack([src_p, src_p + _N]).reshape(_NC * _NRC, _CHUNK)
    scatter_idx = dst_p.reshape(_NRC, _CHUNK)
    deg_idx = jnp.concatenate(
        [edge_index, jnp.full((2, pad), _DUMMY, jnp.int32)], axis=1
    ).reshape(_NC * _NRC, _CHUNK)

    deg = _deg_kernel(deg_idx).reshape(_NC, _AGG_ROWS, _DEGW)
    cnt_out = deg[0, :_N, 0:1]
    cnt_in = deg[1, :_N, 0:1]

    x1 = _mm1(h, cnt_out, W1).reshape(_NC * _N, _HALF)
    agg1 = _agg_kernel(x1, gather_idx, scatter_idx)
    x2 = _mm2(agg1, cnt_in, cnt_out, b1.reshape(1, _D), W2).reshape(
        _NC * _N, _HALF)
    agg2 = _agg_kernel(x2, gather_idx, scatter_idx)
    return _final(agg2, cnt_in, b2.reshape(1, _D))


# speedup vs baseline: 2.9898x; 2.9898x over previous
import functools

import jax
import jax.numpy as jnp
from jax import lax
from jax.experimental import pallas as pl
from jax.experimental.pallas import tpu as pltpu
from jax.experimental.pallas import tpu_sc as plsc

_N = 10000
_E = 160000
_D = 256
_HALF = 128
_NC = 2
_NS = 16
_CHUNK = 128
_EPAD = 2048 * 80
_CPT = _EPAD // (_NS * _CHUNK)
_NRC = _EPAD // _CHUNK
_AC = 64
_ACPT = _EPAD // (_NS * _AC)
_ANRC = _EPAD // _AC
_ASTAGE = 32
_AGG_ROWS = 10240
_DUMMY = _N
_DEGW = 16
_BN = 1000


def _make_mesh():
    return plsc.VectorSubcoreMesh(
        core_axis_name="c", subcore_axis_name="s",
        num_cores=_NC, num_subcores=_NS)


_DSTAGE = 16


@functools.partial(
    pl.kernel,
    out_type=jax.ShapeDtypeStruct((_NC * _NS * _AGG_ROWS,), jnp.float32),
    mesh=_make_mesh(),
    scratch_types=[
        pltpu.VMEM((_DSTAGE, _CHUNK), jnp.int32),
        pltpu.VMEM((_AGG_ROWS,), jnp.float32),
    ],
    compiler_params=pltpu.CompilerParams(needs_layout_passes=False),
)
def _deg_kernel(eidx_hbm, deg_hbm, didx, hist):
    c = lax.axis_index("c")
    t = lax.axis_index("s")
    zero = jnp.zeros((16,), jnp.float32)

    def zero_body(k, carry):
        hist[pl.ds(k * 16, 16)] = zero
        return carry

    lax.fori_loop(0, _AGG_ROWS // 16, zero_body, 0)

    def stage_body(st, carry):
        pltpu.sync_copy(
            eidx_hbm.at[pl.ds(c * _NRC + t * _CPT + st * _DSTAGE, _DSTAGE)],
            didx)

        def row_body(g, carry2):
            for j in range(_CHUNK // 16):
                idx = didx[g, pl.ds(j * 16, 16)]
                cnt, last = plsc.scan_count(idx)
                plsc.addupdate_scatter(
                    hist, [idx], cnt.astype(jnp.float32), mask=last)
            return carry2

        lax.fori_loop(0, _DSTAGE, row_body, 0)
        return carry

    lax.fori_loop(0, _CPT // _DSTAGE, stage_body, 0)
    pltpu.sync_copy(
        hist, deg_hbm.at[pl.ds((c * _NS + t) * _AGG_ROWS, _AGG_ROWS)])


@functools.partial(
    pl.kernel,
    out_type=jax.ShapeDtypeStruct((_NC * _N, _HALF), jnp.float32),
    mesh=_make_mesh(),
    scratch_types=[
        pltpu.VMEM((_ASTAGE, _AC), jnp.int32),
        pltpu.VMEM((_ASTAGE, _AC), jnp.int32),
        pltpu.VMEM((_AC, _HALF), jnp.float32),
        pltpu.VMEM((_AC, _HALF), jnp.float32),
        pltpu.VMEM_SHARED((_AGG_ROWS, _HALF), jnp.float32),
        pltpu.SemaphoreType.DMA,
        pltpu.SemaphoreType.DMA,
    ],
)
def _agg_kernel(x_hbm, gidx_hbm, sidx_hbm, out_hbm,
                gidx, didx, buf0, buf1, agg, sem0, sem1):
    c = lax.axis_index("c")
    t = lax.axis_index("s")

    zero = jnp.zeros((16,), jnp.float32)
    for i in range(_AC):
        for j in range(_HALF // 16):
            buf0[i, pl.ds(j * 16, 16)] = zero
    rpt = _AGG_ROWS // _NS

    def zero_body(k, carry):
        pltpu.sync_copy(buf0, agg.at[pl.ds(t * rpt + k * _AC, _AC)])
        return carry

    lax.fori_loop(0, rpt // _AC, zero_body, 0)
    plsc.subcore_barrier()

    def gather(g, buf, sem):
        return pltpu.make_async_copy(x_hbm.at[gidx.at[g]], buf, sem)

    def stage_body(st, carry):
        pltpu.sync_copy(
            gidx_hbm.at[pl.ds(c * _ANRC + t * _ACPT + st * _ASTAGE, _ASTAGE)],
            gidx)
        pltpu.sync_copy(
            sidx_hbm.at[pl.ds(t * _ACPT + st * _ASTAGE, _ASTAGE)], didx)
        gather(0, buf0, sem0).start()

        def body(i, carry2):
            g0 = 2 * i
            g1 = 2 * i + 1
            gather(g0, buf0, sem0).wait()
            gather(g1, buf1, sem1).start()
            pltpu.sync_copy(buf0, agg.at[didx.at[g0]], add=True)
            gather(g1, buf1, sem1).wait()

            @pl.when(i < _ASTAGE // 2 - 1)
            def _():
                gather(g1 + 1, buf0, sem0).start()

            pltpu.sync_copy(buf1, agg.at[didx.at[g1]], add=True)
            return carry2

        lax.fori_loop(0, _ASTAGE // 2, body, 0)
        return carry

    lax.fori_loop(0, _ACPT // _ASTAGE, stage_body, 0)
    plsc.subcore_barrier()
    @pl.when(t < _NS - 1)
    def _():
        pltpu.sync_copy(agg.at[pl.ds(t * 624, 624)],
                        out_hbm.at[pl.ds(c * _N + t * 624, 624)])

    @pl.when(t == _NS - 1)
    def _():
        pltpu.sync_copy(agg.at[pl.ds(15 * 624, 640)],
                        out_hbm.at[pl.ds(c * _N + 15 * 624, 640)])


def _scales(degp):
    def body(p_ref, o_ref):
        cnt = jnp.sum(p_ref[0], axis=0)
        s = lax.rsqrt(jnp.maximum(cnt, 1.0))
        o_ref[0] = s.reshape(_AGG_ROWS, 1)

    return pl.pallas_call(
        body,
        grid=(_NC,),
        in_specs=[pl.BlockSpec((1, _NS, _AGG_ROWS), lambda i: (i, 0, 0))],
        out_specs=pl.BlockSpec((1, _AGG_ROWS, 1), lambda i: (i, 0, 0)),
        out_shape=jax.ShapeDtypeStruct((_NC, _AGG_ROWS, 1), jnp.float32),
    )(degp)


def _mm1(h, s_out, W1):
    def body(h_ref, c_ref, w_ref, o_ref):
        r = jnp.dot(h_ref[...] * c_ref[...], w_ref[...],
                    preferred_element_type=jnp.float32)
        o_ref[0] = r[:, :_HALF]
        o_ref[1] = r[:, _HALF:]

    return pl.pallas_call(
        body,
        grid=(_N // _BN,),
        in_specs=[
            pl.BlockSpec((_BN, _D), lambda i: (i, 0)),
            pl.BlockSpec((_BN, 1), lambda i: (i, 0)),
            pl.BlockSpec((_D, _D), lambda i: (0, 0)),
        ],
        out_specs=pl.BlockSpec((_NC, _BN, _HALF), lambda i: (0, i, 0)),
        out_shape=jax.ShapeDtypeStruct((_NC, _N, _HALF), jnp.float32),
    )(h, s_out, W1)


def _mm2(agg1, s_in, s_out, b1, W2):
    nb = _N // _BN

    def body(lo_ref, hi_ref, ci_ref, co_ref, b_ref, w_ref, o_ref):
        a = jnp.concatenate([lo_ref[...], hi_ref[...]], axis=1)
        h1 = jnp.maximum(a * ci_ref[...] + b_ref[...], 0.0)
        r = jnp.dot(h1 * co_ref[...], w_ref[...],
                    preferred_element_type=jnp.float32)
        o_ref[0] = r[:, :_HALF]
        o_ref[1] = r[:, _HALF:]

    return pl.pallas_call(
        body,
        grid=(nb,),
        in_specs=[
            pl.BlockSpec((_BN, _HALF), lambda i: (i, 0)),
            pl.BlockSpec((_BN, _HALF), lambda i: (nb + i, 0)),
            pl.BlockSpec((_BN, 1), lambda i: (i, 0)),
            pl.BlockSpec((_BN, 1), lambda i: (i, 0)),
            pl.BlockSpec((1, _D), lambda i: (0, 0)),
            pl.BlockSpec((_D, _D), lambda i: (0, 0)),
        ],
        out_specs=pl.BlockSpec((_NC, _BN, _HALF), lambda i: (0, i, 0)),
        out_shape=jax.ShapeDtypeStruct((_NC, _N, _HALF), jnp.float32),
    )(agg1, agg1, s_in, s_out, b1, W2)


def _final(agg2, s_in, b2):
    nb = _N // _BN

    def body(lo_ref, hi_ref, ci_ref, b_ref, o_ref):
        a = jnp.concatenate([lo_ref[...], hi_ref[...]], axis=1)
        o_ref[...] = a * ci_ref[...] + b_ref[...]

    return pl.pallas_call(
        body,
        grid=(nb,),
        in_specs=[
            pl.BlockSpec((_BN, _HALF), lambda i: (i, 0)),
            pl.BlockSpec((_BN, _HALF), lambda i: (nb + i, 0)),
            pl.BlockSpec((_BN, 1), lambda i: (i, 0)),
            pl.BlockSpec((1, _D), lambda i: (0, 0)),
        ],
        out_specs=pl.BlockSpec((_BN, _D), lambda i: (i, 0)),
        out_shape=jax.ShapeDtypeStruct((_N, _D), jnp.float32),
    )(agg2, agg2, s_in, b2)


def kernel(h, edge_index, W1, b1, W2, b2):
    src = edge_index[0]
    dst = edge_index[1]
    pad = _EPAD - _E
    src_p = jnp.concatenate([src, jnp.zeros((pad,), jnp.int32)])
    dst_p = jnp.concatenate([dst, jnp.full((pad,), _DUMMY, jnp.int32)])
    gather_idx = jnp.stack([src_p, src_p + _N]).reshape(_NC * _ANRC, _AC)
    scatter_idx = dst_p.reshape(_ANRC, _AC)
    deg_idx = jnp.concatenate(
        [edge_index, jnp.full((2, pad), _DUMMY, jnp.int32)], axis=1
    ).reshape(_NC * _NRC, _CHUNK)

    degp = _deg_kernel(deg_idx).reshape(_NC, _NS, _AGG_ROWS)
    scales = _scales(degp)
    s_out = scales[0, :_N]
    s_in = scales[1, :_N]

    x1 = _mm1(h, s_out, W1).reshape(_NC * _N, _HALF)
    agg1 = _agg_kernel(x1, gather_idx, scatter_idx)
    x2 = _mm2(agg1, s_in, s_out, b1.reshape(1, _D), W2).reshape(
        _NC * _N, _HALF)
    agg2 = _agg_kernel(x2, gather_idx, scatter_idx)
    return _final(agg2, s_in, b2.reshape(1, _D))


# speedup vs baseline: 3.3962x; 1.1359x over previous
import functools

import jax
import jax.numpy as jnp
from jax import lax
from jax.experimental import pallas as pl
from jax.experimental.pallas import tpu as pltpu
from jax.experimental.pallas import tpu_sc as plsc

_N = 10000
_E = 160000
_D = 256
_HALF = 128
_NC = 2
_NS = 16
_CHUNK = 128
_EPAD = 2048 * 80
_CPT = _EPAD // (_NS * _CHUNK)
_NRC = _EPAD // _CHUNK
_AC = 128
_ACPT = _EPAD // (_NS * _AC)
_ANRC = _EPAD // _AC
_ASTAGE = 8
_AGG_ROWS = 10240
_DUMMY = _N
_DEGW = 16
_BN = 1000


def _make_mesh():
    return plsc.VectorSubcoreMesh(
        core_axis_name="c", subcore_axis_name="s",
        num_cores=_NC, num_subcores=_NS)


_DSTAGE = 16


@functools.partial(
    pl.kernel,
    out_type=jax.ShapeDtypeStruct((_NC * _NS * _AGG_ROWS,), jnp.float32),
    mesh=_make_mesh(),
    scratch_types=[
        pltpu.VMEM((_DSTAGE, _CHUNK), jnp.int32),
        pltpu.VMEM((_AGG_ROWS,), jnp.float32),
    ],
    compiler_params=pltpu.CompilerParams(needs_layout_passes=False),
)
def _deg_kernel(eidx_hbm, deg_hbm, didx, hist):
    c = lax.axis_index("c")
    t = lax.axis_index("s")
    zero = jnp.zeros((16,), jnp.float32)

    def zero_body(k, carry):
        hist[pl.ds(k * 16, 16)] = zero
        return carry

    lax.fori_loop(0, _AGG_ROWS // 16, zero_body, 0)

    def stage_body(st, carry):
        pltpu.sync_copy(
            eidx_hbm.at[pl.ds(c * _NRC + t * _CPT + st * _DSTAGE, _DSTAGE)],
            didx)

        def row_body(g, carry2):
            for j in range(_CHUNK // 16):
                idx = didx[g, pl.ds(j * 16, 16)]
                cnt, last = plsc.scan_count(idx)
                plsc.addupdate_scatter(
                    hist, [idx], cnt.astype(jnp.float32), mask=last)
            return carry2

        lax.fori_loop(0, _DSTAGE, row_body, 0)
        return carry

    lax.fori_loop(0, _CPT // _DSTAGE, stage_body, 0)
    pltpu.sync_copy(
        hist, deg_hbm.at[pl.ds((c * _NS + t) * _AGG_ROWS, _AGG_ROWS)])


@functools.partial(
    pl.kernel,
    out_type=jax.ShapeDtypeStruct((_NC * _N, _HALF), jnp.float32),
    mesh=_make_mesh(),
    scratch_types=[
        pltpu.VMEM((_ASTAGE, _AC), jnp.int32),
        pltpu.VMEM((_ASTAGE, _AC), jnp.int32),
        pltpu.VMEM((_AC, _HALF), jnp.float32),
        pltpu.VMEM((_AC, _HALF), jnp.float32),
        pltpu.VMEM_SHARED((_AGG_ROWS, _HALF), jnp.float32),
        pltpu.SemaphoreType.DMA,
        pltpu.SemaphoreType.DMA,
    ],
)
def _agg_kernel(x_hbm, gidx_hbm, sidx_hbm, out_hbm,
                gidx, didx, buf0, buf1, agg, sem0, sem1):
    c = lax.axis_index("c")
    t = lax.axis_index("s")

    zero = jnp.zeros((16,), jnp.float32)
    for i in range(_AC):
        for j in range(_HALF // 16):
            buf0[i, pl.ds(j * 16, 16)] = zero
    rpt = _AGG_ROWS // _NS

    def zero_body(k, carry):
        pltpu.sync_copy(buf0, agg.at[pl.ds(t * rpt + k * _AC, _AC)])
        return carry

    lax.fori_loop(0, rpt // _AC, zero_body, 0)
    plsc.subcore_barrier()

    def gather(g, buf, sem):
        return pltpu.make_async_copy(x_hbm.at[gidx.at[g]], buf, sem)

    def stage_body(st, carry):
        pltpu.sync_copy(
            gidx_hbm.at[pl.ds(c * _ANRC + t * _ACPT + st * _ASTAGE, _ASTAGE)],
            gidx)
        pltpu.sync_copy(
            sidx_hbm.at[pl.ds(t * _ACPT + st * _ASTAGE, _ASTAGE)], didx)
        gather(0, buf0, sem0).start()

        def body(i, carry2):
            g0 = 2 * i
            g1 = 2 * i + 1
            gather(g0, buf0, sem0).wait()
            gather(g1, buf1, sem1).start()
            pltpu.sync_copy(buf0, agg.at[didx.at[g0]], add=True)
            gather(g1, buf1, sem1).wait()

            @pl.when(i < _ASTAGE // 2 - 1)
            def _():
                gather(g1 + 1, buf0, sem0).start()

            pltpu.sync_copy(buf1, agg.at[didx.at[g1]], add=True)
            return carry2

        lax.fori_loop(0, _ASTAGE // 2, body, 0)
        return carry

    lax.fori_loop(0, _ACPT // _ASTAGE, stage_body, 0)
    plsc.subcore_barrier()
    @pl.when(t < _NS - 1)
    def _():
        pltpu.sync_copy(agg.at[pl.ds(t * 624, 624)],
                        out_hbm.at[pl.ds(c * _N + t * 624, 624)])

    @pl.when(t == _NS - 1)
    def _():
        pltpu.sync_copy(agg.at[pl.ds(15 * 624, 640)],
                        out_hbm.at[pl.ds(c * _N + 15 * 624, 640)])


def _scales(degp):
    def body(p_ref, o_ref):
        cnt = jnp.sum(p_ref[0], axis=0)
        s = lax.rsqrt(jnp.maximum(cnt, 1.0))
        o_ref[0] = s.reshape(_AGG_ROWS, 1)

    return pl.pallas_call(
        body,
        grid=(_NC,),
        in_specs=[pl.BlockSpec((1, _NS, _AGG_ROWS), lambda i: (i, 0, 0))],
        out_specs=pl.BlockSpec((1, _AGG_ROWS, 1), lambda i: (i, 0, 0)),
        out_shape=jax.ShapeDtypeStruct((_NC, _AGG_ROWS, 1), jnp.float32),
    )(degp)


def _mm1(h, s_out, W1):
    def body(h_ref, c_ref, w_ref, o_ref):
        r = jnp.dot(h_ref[...] * c_ref[...], w_ref[...],
                    preferred_element_type=jnp.float32)
        o_ref[0] = r[:, :_HALF]
        o_ref[1] = r[:, _HALF:]

    return pl.pallas_call(
        body,
        grid=(_N // _BN,),
        in_specs=[
            pl.BlockSpec((_BN, _D), lambda i: (i, 0)),
            pl.BlockSpec((_BN, 1), lambda i: (i, 0)),
            pl.BlockSpec((_D, _D), lambda i: (0, 0)),
        ],
        out_specs=pl.BlockSpec((_NC, _BN, _HALF), lambda i: (0, i, 0)),
        out_shape=jax.ShapeDtypeStruct((_NC, _N, _HALF), jnp.float32),
    )(h, s_out, W1)


def _mm2(agg1, s_in, s_out, b1, W2):
    nb = _N // _BN

    def body(lo_ref, hi_ref, ci_ref, co_ref, b_ref, w_ref, o_ref):
        a = jnp.concatenate([lo_ref[...], hi_ref[...]], axis=1)
        h1 = jnp.maximum(a * ci_ref[...] + b_ref[...], 0.0)
        r = jnp.dot(h1 * co_ref[...], w_ref[...],
                    preferred_element_type=jnp.float32)
        o_ref[0] = r[:, :_HALF]
        o_ref[1] = r[:, _HALF:]

    return pl.pallas_call(
        body,
        grid=(nb,),
        in_specs=[
            pl.BlockSpec((_BN, _HALF), lambda i: (i, 0)),
            pl.BlockSpec((_BN, _HALF), lambda i: (nb + i, 0)),
            pl.BlockSpec((_BN, 1), lambda i: (i, 0)),
            pl.BlockSpec((_BN, 1), lambda i: (i, 0)),
            pl.BlockSpec((1, _D), lambda i: (0, 0)),
            pl.BlockSpec((_D, _D), lambda i: (0, 0)),
        ],
        out_specs=pl.BlockSpec((_NC, _BN, _HALF), lambda i: (0, i, 0)),
        out_shape=jax.ShapeDtypeStruct((_NC, _N, _HALF), jnp.float32),
    )(agg1, agg1, s_in, s_out, b1, W2)


def _final(agg2, s_in, b2):
    nb = _N // _BN

    def body(lo_ref, hi_ref, ci_ref, b_ref, o_ref):
        a = jnp.concatenate([lo_ref[...], hi_ref[...]], axis=1)
        o_ref[...] = a * ci_ref[...] + b_ref[...]

    return pl.pallas_call(
        body,
        grid=(nb,),
        in_specs=[
            pl.BlockSpec((_BN, _HALF), lambda i: (i, 0)),
            pl.BlockSpec((_BN, _HALF), lambda i: (nb + i, 0)),
            pl.BlockSpec((_BN, 1), lambda i: (i, 0)),
            pl.BlockSpec((1, _D), lambda i: (0, 0)),
        ],
        out_specs=pl.BlockSpec((_BN, _D), lambda i: (i, 0)),
        out_shape=jax.ShapeDtypeStruct((_N, _D), jnp.float32),
    )(agg2, agg2, s_in, b2)


def kernel(h, edge_index, W1, b1, W2, b2):
    src = edge_index[0]
    dst = edge_index[1]
    pad = _EPAD - _E
    src_p = jnp.concatenate([src, jnp.zeros((pad,), jnp.int32)])
    dst_p = jnp.concatenate([dst, jnp.full((pad,), _DUMMY, jnp.int32)])
    gather_idx = jnp.stack([src_p, src_p + _N]).reshape(_NC * _ANRC, _AC)
    scatter_idx = dst_p.reshape(_ANRC, _AC)
    deg_idx = jnp.concatenate(
        [edge_index, jnp.full((2, pad), _DUMMY, jnp.int32)], axis=1
    ).reshape(_NC * _NRC, _CHUNK)

    degp = _deg_kernel(deg_idx).reshape(_NC, _NS, _AGG_ROWS)
    scales = _scales(degp)
    s_out = scales[0, :_N]
    s_in = scales[1, :_N]

    x1 = _mm1(h, s_out, W1).reshape(_NC * _N, _HALF)
    agg1 = _agg_kernel(x1, gather_idx, scatter_idx)
    x2 = _mm2(agg1, s_in, s_out, b1.reshape(1, _D), W2).reshape(
        _NC * _N, _HALF)
    agg2 = _agg_kernel(x2, gather_idx, scatter_idx)
    return _final(agg2, s_in, b2.reshape(1, _D))
